# baseline (speedup 1.0000x reference)
import functools

import jax
import jax.numpy as jnp
from jax import lax
from jax.experimental import pallas as pl
from jax.experimental.pallas import tpu as pltpu
from jax.experimental.pallas import tpu_sc as plsc

B_, P_, A_, D_ = 512, 256, 196, 128
NM, NU = 192, 64
BS = 16
NMT = B_ * NM
NUT = B_ * NU
NC, NS = 2, 16
NW = NC * NS
CH = 256
ME_PW = NMT // NW
UP_PW = NUT // NW


@functools.cache
def _make_sc_gather():
    mesh = plsc.VectorSubcoreMesh(core_axis_name="c", subcore_axis_name="s")

    @functools.partial(
        pl.kernel,
        mesh=mesh,
        out_type=(
            jax.ShapeDtypeStruct((NMT, D_), jnp.float32),
            jax.ShapeDtypeStruct((NUT, D_), jnp.float32),
        ),
        scratch_types=[
            pltpu.VMEM((CH,), jnp.int32),
            pltpu.VMEM((CH,), jnp.int32),
            pltpu.VMEM((CH, D_), jnp.float32),
            pltpu.VMEM((CH, D_), jnp.float32),
            pltpu.SemaphoreType.DMA,
            pltpu.SemaphoreType.DMA,
            pltpu.SemaphoreType.DMA,
        ],
    )
    def _sc_gather(pp_hbm, pos_hbm, mflat_hbm, uflat_hbm,
                   me_hbm, up_hbm, idx0, idx1, buf0, buf1,
                   semg, semo0, semo1):
        wid = lax.axis_index("s") * NC + lax.axis_index("c")
        idxs = (idx0, idx1)
        bufs = (buf0, buf1)
        semo = (semo0, semo1)
        sched = []
        for c in range(ME_PW // CH):
            sched.append((mflat_hbm, pp_hbm, me_hbm, wid * ME_PW + c * CH))
        for c in range(UP_PW // CH):
            sched.append((uflat_hbm, pos_hbm, up_hbm, wid * UP_PW + c * CH))
        outcp = [None, None]
        for c, (iflat, tbl, out, base) in enumerate(sched):
            bsl = c % 2
            if outcp[bsl] is not None:
                outcp[bsl].wait()
            pltpu.sync_copy(iflat.at[pl.ds(base, CH)], idxs[bsl])
            pltpu.async_copy(tbl.at[idxs[bsl]], bufs[bsl], semg).wait()
            cp = pltpu.make_async_copy(bufs[bsl], out.at[pl.ds(base, CH)],
                                       semo[bsl])
            cp.start()
            outcp[bsl] = cp
        for cp in outcp:
            cp.wait()

    return _sc_gather


def _mvec_body(mtok_ref, W_ref, b_ref, pos_ref, pp_ref):
    mv = jnp.dot(mtok_ref[...], W_ref[...],
                 preferred_element_type=jnp.float32) + b_ref[...]
    pp_ref[...] = pos_ref[...] + mv


def _enc_body(idx_ref, patches_ref, W_ref, b_ref, pos_ref, ue_ref):
    Wb = W_ref[...].astype(jnp.bfloat16)
    bvec = b_ref[...]
    pos = pos_ref[...].astype(jnp.bfloat16)
    for s in range(BS):
        idxs = idx_ref[s, NM:]
        idx_col = jnp.reshape(idxs, (NU, 1))
        du = (idx_col == lax.broadcasted_iota(jnp.int32, (NU, P_), 1)
              ).astype(jnp.bfloat16)
        upos = jnp.dot(du, pos, preferred_element_type=jnp.float32)
        gp = jnp.dot(du, patches_ref[s].astype(jnp.bfloat16),
                     preferred_element_type=jnp.float32).astype(jnp.bfloat16)
        ue_ref[s] = jnp.dot(gp, Wb, preferred_element_type=jnp.float32
                            ) + bvec + upos


def kernel(patches, W, b, pos_table, mask_token, rand_uniform):
    idx_sorted = jnp.argsort(rand_uniform, axis=-1).astype(jnp.int32)
    mask_indices = idx_sorted[:, :NM]
    unmask_indices = idx_sorted[:, NM:]
    b2 = b.reshape(1, D_)

    pos_plus = pl.pallas_call(
        _mvec_body,
        out_shape=jax.ShapeDtypeStruct((P_, D_), jnp.float32),
    )(mask_token, W, b2, pos_table)

    me_flat, up_flat = _make_sc_gather()(
        pos_plus, pos_table, mask_indices.reshape(NMT),
        unmask_indices.reshape(NUT))

    ue = pl.pallas_call(
        _enc_body,
        grid=(B_ // BS,),
        in_specs=[
            pl.BlockSpec((BS, P_), lambda i: (i, 0)),
            pl.BlockSpec((BS, P_, A_), lambda i: (i, 0, 0)),
            pl.BlockSpec((A_, D_), lambda i: (0, 0)),
            pl.BlockSpec((1, D_), lambda i: (0, 0)),
            pl.BlockSpec((P_, D_), lambda i: (0, 0)),
        ],
        out_specs=pl.BlockSpec((BS, NU, D_), lambda i: (i, 0, 0)),
        out_shape=jax.ShapeDtypeStruct((B_, NU, D_), jnp.float32),
        compiler_params=pltpu.CompilerParams(
            dimension_semantics=("parallel",),
        ),
    )(idx_sorted, patches, W, b2, pos_table)

    return (ue, me_flat.reshape(B_, NM, D_), up_flat.reshape(B_, NU, D_),
            mask_indices, unmask_indices)


# speedup vs baseline: 1.4461x; 1.4461x over previous
import jax
import jax.numpy as jnp
from jax import lax
from jax.experimental import pallas as pl
from jax.experimental.pallas import tpu as pltpu

B_, P_, A_, D_ = 512, 256, 196, 128
NM, NU = 192, 64
BS = 16


def _enc_body(idx_ref, patches_ref, W_ref, b_ref, pos_ref, mtok_ref,
              ue_ref, me_ref, up_ref):
    W = W_ref[...]
    Wb = W.astype(jnp.bfloat16)
    bvec = b_ref[...]
    pos = pos_ref[...].astype(jnp.bfloat16)
    mvec = jnp.dot(mtok_ref[...], W, preferred_element_type=jnp.float32) + bvec
    for s in range(BS):
        idx_col = idx_ref[0, :, s:s + 1]
        D = (idx_col == lax.broadcasted_iota(jnp.int32, (P_, P_), 1)
             ).astype(jnp.bfloat16)
        du = D[NM:, :]
        dm = D[:NM, :]
        pb = patches_ref[s].astype(jnp.bfloat16)
        gp = jnp.dot(du, pb, preferred_element_type=jnp.float32
                     ).astype(jnp.bfloat16)
        upos = jnp.dot(du, pos, preferred_element_type=jnp.float32)
        mpos = jnp.dot(dm, pos, preferred_element_type=jnp.float32)
        ue_ref[s] = jnp.dot(gp, Wb, preferred_element_type=jnp.float32) + bvec + upos
        up_ref[s] = upos
        me_ref[s] = mvec + mpos


def kernel(patches, W, b, pos_table, mask_token, rand_uniform):
    idx_sorted = jnp.argsort(rand_uniform, axis=-1).astype(jnp.int32)
    grid = (B_ // BS,)
    out_shapes = (
        jax.ShapeDtypeStruct((B_, NU, D_), jnp.float32),
        jax.ShapeDtypeStruct((B_, NM, D_), jnp.float32),
        jax.ShapeDtypeStruct((B_, NU, D_), jnp.float32),
    )
    ue, me, up = pl.pallas_call(
        _enc_body,
        grid=grid,
        in_specs=[
            pl.BlockSpec((1, P_, BS), lambda i: (i, 0, 0)),
            pl.BlockSpec((BS, P_, A_), lambda i: (i, 0, 0)),
            pl.BlockSpec((A_, D_), lambda i: (0, 0)),
            pl.BlockSpec((1, D_), lambda i: (0, 0)),
            pl.BlockSpec((P_, D_), lambda i: (0, 0)),
            pl.BlockSpec((1, A_), lambda i: (0, 0)),
        ],
        out_specs=(
            pl.BlockSpec((BS, NU, D_), lambda i: (i, 0, 0)),
            pl.BlockSpec((BS, NM, D_), lambda i: (i, 0, 0)),
            pl.BlockSpec((BS, NU, D_), lambda i: (i, 0, 0)),
        ),
        out_shape=out_shapes,
        compiler_params=pltpu.CompilerParams(
            dimension_semantics=("parallel",),
        ),
    )(idx_sorted.reshape(B_ // BS, BS, P_).swapaxes(1, 2), patches, W,
      b.reshape(1, D_), pos_table, mask_token)
    mask_indices = idx_sorted[:, :NM]
    unmask_indices = idx_sorted[:, NM:]
    return ue, me, up, mask_indices, unmask_indices
